# baseline (speedup 1.0000x reference)
import functools

import jax
import jax.numpy as jnp
from jax import lax
from jax.experimental import pallas as pl
from jax.experimental.pallas import tpu as pltpu
from jax.experimental.pallas import tpu_sc as plsc

NUM_ENTITIES = 1000000
EMBED_DIM = 64
BATCH = 16384
FIELDS = 26

NC = 2
NS = 16
NW = NC * NS

B_PER_W = BATCH // NW
ROWS = 256
NSPLIT = B_PER_W // ROWS
NCHUNK = FIELDS * NSPLIT
NBUF = 3
DELAY = NBUF - 1


@functools.partial(
    pl.kernel,
    out_type=jax.ShapeDtypeStruct((BATCH, FIELDS, 2 * EMBED_DIM), jnp.float32),
    mesh=plsc.VectorSubcoreMesh(core_axis_name="c", subcore_axis_name="s"),
    scratch_types=[
        pltpu.VMEM((FIELDS * B_PER_W,), jnp.int32),
        [pltpu.VMEM((ROWS, 2 * EMBED_DIM), jnp.float32) for _ in range(NBUF)],
        [pltpu.SemaphoreType.DMA for _ in range(NBUF)],
        [pltpu.SemaphoreType.DMA for _ in range(NBUF)],
        pltpu.SemaphoreType.DMA,
    ],
    compiler_params=pltpu.CompilerParams(use_tc_tiling_on_sc=True),
)
def _gather_kernel(idx_hbm, pairs_hbm, out_hbm, idx_v, rows, gsem, wsem, isem):
    wid = lax.axis_index("s") * NC + lax.axis_index("c")
    b0 = wid * B_PER_W

    for f in range(FIELDS):
        pltpu.make_async_copy(
            idx_hbm.at[pl.ds(f * BATCH + b0, B_PER_W)],
            idx_v.at[pl.ds(f * B_PER_W, B_PER_W)],
            isem,
        ).start()
    for f in range(FIELDS):
        pltpu.make_async_copy(
            idx_hbm.at[pl.ds(f * BATCH + b0, B_PER_W)],
            idx_v.at[pl.ds(f * B_PER_W, B_PER_W)],
            isem,
        ).wait()

    def start_gather(c):
        s = c % NBUF
        pltpu.make_async_copy(
            pairs_hbm.at[idx_v.at[pl.ds(c * ROWS, ROWS)]], rows[s], gsem[s]
        ).start()

    def finish_and_writeback(c):
        s = c % NBUF
        f, h = c // NSPLIT, c % NSPLIT
        pltpu.make_async_copy(
            pairs_hbm.at[idx_v.at[pl.ds(c * ROWS, ROWS)]], rows[s], gsem[s]
        ).wait()
        pltpu.make_async_copy(
            rows[s], out_hbm.at[pl.ds(b0 + h * ROWS, ROWS), f], wsem[s]
        ).start()

    def wait_writeback(c):
        s = c % NBUF
        f, h = c // NSPLIT, c % NSPLIT
        pltpu.make_async_copy(
            rows[s], out_hbm.at[pl.ds(b0 + h * ROWS, ROWS), f], wsem[s]
        ).wait()

    for c in range(NCHUNK + DELAY):
        if c < NCHUNK:
            if c >= NBUF:
                wait_writeback(c - NBUF)
            start_gather(c)
        if c >= DELAY:
            finish_and_writeback(c - DELAY)
    for c in range(max(NCHUNK - NBUF, 0), NCHUNK):
        wait_writeback(c)


def kernel(indices, entity_embeddings):
    flat_idx = jnp.transpose(indices).astype(jnp.int32).reshape(FIELDS * BATCH)
    pair_idx = flat_idx // 2
    half = flat_idx % 2
    pairs = entity_embeddings.reshape(NUM_ENTITIES // 2, 2 * EMBED_DIM)
    out = _gather_kernel(pair_idx, pairs)
    hsel = jnp.transpose(half.reshape(FIELDS, BATCH))[:, :, None] == 1
    return jnp.where(hsel, out[:, :, EMBED_DIM:], out[:, :, :EMBED_DIM])


# speedup vs baseline: 1.1543x; 1.1543x over previous
import functools

import jax
import jax.numpy as jnp
from jax import lax
from jax.experimental import pallas as pl
from jax.experimental.pallas import tpu as pltpu
from jax.experimental.pallas import tpu_sc as plsc

NUM_ENTITIES = 1000000
EMBED_DIM = 64
BATCH = 16384
FIELDS = 26

NC = 2
NS = 16
NW = NC * NS

B_PER_W = BATCH // NW
N_IDX = B_PER_W * FIELDS
CB = 16
ROWS = CB * FIELDS
NCHUNK = B_PER_W // CB
NBUF = 3
DELAY = NBUF - 1
L = 16


@functools.partial(
    pl.kernel,
    out_type=jax.ShapeDtypeStruct((BATCH * FIELDS, EMBED_DIM), jnp.float32),
    mesh=plsc.VectorSubcoreMesh(core_axis_name="c", subcore_axis_name="s"),
    scratch_types=[
        pltpu.VMEM((N_IDX,), jnp.int32),
        pltpu.VMEM((N_IDX,), jnp.int32),
        [pltpu.VMEM((ROWS, EMBED_DIM), jnp.float32) for _ in range(NBUF)],
        [pltpu.SemaphoreType.DMA for _ in range(NBUF)],
        [pltpu.SemaphoreType.DMA for _ in range(NBUF)],
        pltpu.SemaphoreType.DMA,
    ],
    compiler_params=pltpu.CompilerParams(
        use_tc_tiling_on_sc=False, needs_layout_passes=False
    ),
)
def _gather_kernel(idx_hbm, table_hbm, out_hbm, idx_f, idx_b, rows, gsem,
                   wsem, isem):
    wid = lax.axis_index("s") * NC + lax.axis_index("c")
    b0 = wid * B_PER_W

    for f in range(FIELDS):
        pltpu.make_async_copy(
            idx_hbm.at[pl.ds(f * BATCH + b0, B_PER_W)],
            idx_f.at[pl.ds(f * B_PER_W, B_PER_W)],
            isem,
        ).start()
    for f in range(FIELDS):
        pltpu.make_async_copy(
            idx_hbm.at[pl.ds(f * BATCH + b0, B_PER_W)],
            idx_f.at[pl.ds(f * B_PER_W, B_PER_W)],
            isem,
        ).wait()

    iota = lax.iota(jnp.int32, L)

    def tbody(i, carry):
        m = i * L + iota
        src = (m % FIELDS) * B_PER_W + m // FIELDS
        idx_b[pl.ds(i * L, L)] = plsc.load_gather(idx_f, [src])
        return carry

    lax.fori_loop(0, N_IDX // L, tbody, 0)

    def start_gather(c):
        s = c % NBUF
        pltpu.make_async_copy(
            table_hbm.at[idx_b.at[pl.ds(c * ROWS, ROWS)]], rows[s], gsem[s]
        ).start()

    def finish_and_writeback(c):
        s = c % NBUF
        pltpu.make_async_copy(
            table_hbm.at[idx_b.at[pl.ds(c * ROWS, ROWS)]], rows[s], gsem[s]
        ).wait()
        pltpu.make_async_copy(
            rows[s], out_hbm.at[pl.ds((b0 + c * CB) * FIELDS, ROWS)], wsem[s]
        ).start()

    def wait_writeback(c):
        s = c % NBUF
        pltpu.make_async_copy(
            rows[s], out_hbm.at[pl.ds((b0 + c * CB) * FIELDS, ROWS)], wsem[s]
        ).wait()

    for c in range(NCHUNK + DELAY):
        if c < NCHUNK:
            if c >= NBUF:
                wait_writeback(c - NBUF)
            start_gather(c)
        if c >= DELAY:
            finish_and_writeback(c - DELAY)
    for c in range(max(NCHUNK - NBUF, 0), NCHUNK):
        wait_writeback(c)


def kernel(indices, entity_embeddings):
    flat_idx = jnp.transpose(indices).astype(jnp.int32).reshape(FIELDS * BATCH)
    out = _gather_kernel(flat_idx, entity_embeddings)
    return out.reshape(BATCH, FIELDS, EMBED_DIM)


# speedup vs baseline: 1.2136x; 1.0514x over previous
import functools

import jax
import jax.numpy as jnp
from jax import lax
from jax.experimental import pallas as pl
from jax.experimental.pallas import tpu as pltpu
from jax.experimental.pallas import tpu_sc as plsc

NUM_ENTITIES = 1000000
EMBED_DIM = 64
BATCH = 16384
FIELDS = 26

NC = 2
NS = 16
NW = NC * NS

B_PER_W = BATCH // NW
ROWS = B_PER_W
NBUF = 3
DELAY = NBUF - 1


@functools.partial(
    pl.kernel,
    out_type=jax.ShapeDtypeStruct((FIELDS * BATCH, EMBED_DIM), jnp.float32),
    mesh=plsc.VectorSubcoreMesh(core_axis_name="c", subcore_axis_name="s"),
    scratch_types=[
        pltpu.VMEM((FIELDS * ROWS,), jnp.int32),
        [pltpu.VMEM((ROWS, EMBED_DIM), jnp.float32) for _ in range(NBUF)],
        [pltpu.SemaphoreType.DMA for _ in range(NBUF)],
        [pltpu.SemaphoreType.DMA for _ in range(NBUF)],
        pltpu.SemaphoreType.DMA,
    ],
    compiler_params=pltpu.CompilerParams(use_tc_tiling_on_sc=False),
)
def _gather_kernel(idx_hbm, table_hbm, out_hbm, idx_v, rows, gsem, wsem, isem):
    wid = lax.axis_index("s") * NC + lax.axis_index("c")
    b0 = wid * B_PER_W

    for f in range(FIELDS):
        pltpu.make_async_copy(
            idx_hbm.at[pl.ds(f * BATCH + b0, ROWS)],
            idx_v.at[pl.ds(f * ROWS, ROWS)],
            isem,
        ).start()
    for f in range(FIELDS):
        pltpu.make_async_copy(
            idx_hbm.at[pl.ds(f * BATCH + b0, ROWS)],
            idx_v.at[pl.ds(f * ROWS, ROWS)],
            isem,
        ).wait()

    def start_gather(f):
        s = f % NBUF
        pltpu.make_async_copy(
            table_hbm.at[idx_v.at[pl.ds(f * ROWS, ROWS)]], rows[s], gsem[s]
        ).start()

    def finish_and_writeback(f):
        s = f % NBUF
        pltpu.make_async_copy(
            table_hbm.at[idx_v.at[pl.ds(f * ROWS, ROWS)]], rows[s], gsem[s]
        ).wait()
        pltpu.make_async_copy(
            rows[s], out_hbm.at[pl.ds(f * BATCH + b0, ROWS)], wsem[s]
        ).start()

    def wait_writeback(f):
        s = f % NBUF
        pltpu.make_async_copy(
            rows[s], out_hbm.at[pl.ds(f * BATCH + b0, ROWS)], wsem[s]
        ).wait()

    for f in range(FIELDS + DELAY):
        if f < FIELDS:
            if f >= NBUF:
                wait_writeback(f - NBUF)
            start_gather(f)
        if f >= DELAY:
            finish_and_writeback(f - DELAY)
    for f in range(max(FIELDS - NBUF, 0), FIELDS):
        wait_writeback(f)


def kernel(indices, entity_embeddings):
    flat_idx = jnp.transpose(indices).astype(jnp.int32).reshape(FIELDS * BATCH)
    out = _gather_kernel(flat_idx, entity_embeddings)
    return jnp.transpose(out.reshape(FIELDS, BATCH, EMBED_DIM), (1, 0, 2))
